# baseline (speedup 1.0000x reference)
import functools

import jax
import jax.numpy as jnp
from jax import lax
from jax.experimental import pallas as pl
from jax.experimental.pallas import tpu as pltpu
from jax.experimental.pallas import tpu_sc as plsc

BATCH = 16384
NW = 32
BPW = BATCH // NW
NCHUNK = 4
CHUNK = BPW // NCHUNK

_BCAST_DNUMS = lax.GatherDimensionNumbers(
    offset_dims=(), collapsed_slice_dims=(0,), start_index_map=(0,))


def _bcast(vec, i):
    idx = jnp.full((16, 1), i, dtype=jnp.int32)
    return lax.gather(vec, idx, _BCAST_DNUMS, (1,),
                      mode=lax.GatherScatterMode.PROMISE_IN_BOUNDS)


def _body(user_h, item_h, gu_h, gi_h, mu_h, mi_h, par_h, out_h,
          idx_u, idx_i, idm_u, idm_i, idg_u, idg_i, idx_uf, idx_if,
          g_mu, g_mi, g_gu, g_gi, w_v, part_v, out_v, sem):
    wid = lax.axis_index("s") * 2 + lax.axis_index("c")

    sl = pl.ds(wid * NCHUNK, NCHUNK)
    pltpu.sync_copy(user_h.at[sl], idx_u)
    pltpu.sync_copy(item_h.at[sl], idx_i)
    pltpu.sync_copy(par_h, w_v)

    for j in range(NCHUNK):
        for k in range(CHUNK // 16):
            s = pl.ds(k * 16, 16)
            u = idx_u[j, s]
            i = idx_i[j, s]
            idm_u[j, s] = jnp.right_shift(u, 3)
            idm_i[j, s] = jnp.right_shift(i, 3)
            idg_u[j, s] = jnp.right_shift(u, 4)
            idg_i[j, s] = jnp.right_shift(i, 4)
            f = pl.ds(j * CHUNK + k * 16, 16)
            idx_uf[f] = u
            idx_if[f] = i

    b1v = w_v[pl.ds(48 * 16, 16)]
    b2v = w_v[pl.ds(49 * 16, 16)]
    wlo = w_v[pl.ds(50 * 16, 16)]
    whi = w_v[pl.ds(51 * 16, 16)]
    lane8 = jnp.arange(16, dtype=jnp.int32) & 7
    lane16 = jnp.arange(16, dtype=jnp.int32)
    zeros16 = jnp.zeros((16,), jnp.int32)

    for j in range(NCHUNK):
        c1 = pltpu.async_copy(mu_h.at[idm_u.at[j]], g_mu, sem)
        c2 = pltpu.async_copy(mi_h.at[idm_i.at[j]], g_mi, sem)
        c3 = pltpu.async_copy(gu_h.at[idg_u.at[j]], g_gu, sem)
        c4 = pltpu.async_copy(gi_h.at[idg_i.at[j]], g_gi, sem)
        c1.wait()
        c2.wait()
        c3.wait()
        c4.wait()

        def elem(e, carry, j=j):
            rowv = zeros16 + e
            uv = plsc.load_gather(idx_uf, [zeros16 + (j * CHUNK) + e])
            iv = plsc.load_gather(idx_if, [zeros16 + (j * CHUNK) + e])
            mur = plsc.load_gather(g_mu, [rowv, (uv & 7) * 16 + lane16])
            mir = plsc.load_gather(g_mi, [rowv, (iv & 7) * 16 + lane16])
            acc = b1v
            for i in range(16):
                acc = acc + _bcast(mur, i) * w_v[pl.ds(i * 16, 16)]
            for i in range(16):
                acc = acc + _bcast(mir, i) * w_v[pl.ds((16 + i) * 16, 16)]
            h = jnp.maximum(acc, 0.0)
            acc2 = b2v
            for i in range(16):
                acc2 = acc2 + _bcast(h, i) * w_v[pl.ds((32 + i) * 16, 16)]
            h2 = jnp.maximum(acc2, 0.0)

            gus = plsc.load_gather(g_gu, [rowv, (uv & 15) * 8 + lane8])
            gis = plsc.load_gather(g_gi, [rowv, (iv & 15) * 8 + lane8])

            flat = (j * CHUNK + e) * 16 + lane16
            plsc.store_scatter(part_v, [flat], gus * gis * wlo + h2 * whi)
            return carry

        lax.fori_loop(0, CHUNK, elem, 0)

    blv = w_v[pl.ds(52 * 16, 16)]
    bl0 = blv[0]
    for g in range(BPW // 16):
        rows = (lane16 + g * 16) * 16
        acc = jnp.zeros((16,), jnp.float32) + bl0
        for c in range(16):
            acc = acc + plsc.load_gather(part_v, [rows + c])
        out_v[pl.ds(g * 16, 16)] = 1.0 / (1.0 + jnp.exp(-acc))

    pltpu.sync_copy(out_v, out_h.at[pl.ds(wid * BPW, BPW)])


@jax.jit
def _fused(user2, item2, gu2, gi2, mu2, mi2, params):
    mesh = plsc.VectorSubcoreMesh(core_axis_name="c", subcore_axis_name="s")
    f = functools.partial(
        pl.kernel,
        out_type=jax.ShapeDtypeStruct((BATCH,), jnp.float32),
        mesh=mesh,
        compiler_params=pltpu.CompilerParams(needs_layout_passes=False),
        scratch_types=[
            pltpu.VMEM((NCHUNK, CHUNK), jnp.int32),
            pltpu.VMEM((NCHUNK, CHUNK), jnp.int32),
            pltpu.VMEM((NCHUNK, CHUNK), jnp.int32),
            pltpu.VMEM((NCHUNK, CHUNK), jnp.int32),
            pltpu.VMEM((NCHUNK, CHUNK), jnp.int32),
            pltpu.VMEM((NCHUNK, CHUNK), jnp.int32),
            pltpu.VMEM((BPW,), jnp.int32),
            pltpu.VMEM((BPW,), jnp.int32),
            pltpu.VMEM((CHUNK, 128), jnp.float32),
            pltpu.VMEM((CHUNK, 128), jnp.float32),
            pltpu.VMEM((CHUNK, 128), jnp.float32),
            pltpu.VMEM((CHUNK, 128), jnp.float32),
            pltpu.VMEM((53 * 16,), jnp.float32),
            pltpu.VMEM((BPW * 16,), jnp.float32),
            pltpu.VMEM((BPW,), jnp.float32),
            pltpu.SemaphoreType.DMA,
        ],
    )(_body)
    return f(user2, item2, gu2, gi2, mu2, mi2, params)


def kernel(user, item, gmf_user_emb, gmf_item_emb, mlp_user_emb, mlp_item_emb,
           W1, b1, W2, b2, Wl, bl):
    user2 = user.reshape(128, 128)
    item2 = item.reshape(128, 128)
    gu2 = gmf_user_emb.reshape(-1, 128)
    gi2 = gmf_item_emb.reshape(-1, 128)
    mu2 = mlp_user_emb.reshape(-1, 128)
    mi2 = mlp_item_emb.reshape(-1, 128)
    wl = Wl.reshape(16)
    params = jnp.concatenate([
        W1,
        jnp.pad(W2, ((0, 0), (0, 8))),
        b1.reshape(1, 16),
        jnp.pad(b2, (0, 8)).reshape(1, 16),
        jnp.pad(wl[:8], (0, 8)).reshape(1, 16),
        jnp.pad(wl[8:], (0, 8)).reshape(1, 16),
        jnp.pad(bl, (0, 15)).reshape(1, 16),
    ], axis=0).reshape(-1)
    return _fused(user2, item2, gu2, gi2, mu2, mi2, params)


# speedup vs baseline: 5.6804x; 5.6804x over previous
import functools

import jax
import jax.numpy as jnp
from jax import lax
from jax.experimental import pallas as pl
from jax.experimental.pallas import tpu as pltpu
from jax.experimental.pallas import tpu_sc as plsc

BATCH = 16384
NW = 32
BPW = BATCH // NW
NUSERS = 1000000
LAST_TILE = (NUSERS // 128) * 128
LAST_FULL = LAST_TILE - 128

_BCAST_DNUMS = lax.GatherDimensionNumbers(
    offset_dims=(), collapsed_slice_dims=(0,), start_index_map=(0,))


def _bcast(vec, i):
    idx = jnp.full((16, 1), i, dtype=jnp.int32)
    return lax.gather(vec, idx, _BCAST_DNUMS, (1,),
                      mode=lax.GatherScatterMode.PROMISE_IN_BOUNDS)


def _body(user_h, item_h, gu_h, gi_h, mu_h, mi_h, par_h, out_h,
          idx_u, idx_i, mu_v, mi_v, gp_v, w_v, part_v, out_v,
          r_mu, r_mi, r_gu, r_gi, t_mu, t_mi, t_gu, t_gi, sem):
    wid = lax.axis_index("s") * 2 + lax.axis_index("c")

    pltpu.sync_copy(user_h.at[pl.ds(wid * BPW, BPW)], idx_u)
    pltpu.sync_copy(item_h.at[pl.ds(wid * BPW, BPW)], idx_i)
    pltpu.sync_copy(par_h, w_v)
    pltpu.sync_copy(mu_h.at[pl.ds(0, 16), pl.ds(LAST_TILE, 64)], t_mu)
    pltpu.sync_copy(mi_h.at[pl.ds(0, 16), pl.ds(LAST_TILE, 64)], t_mi)
    pltpu.sync_copy(gu_h.at[pl.ds(0, 8), pl.ds(LAST_TILE, 64)], t_gu)
    pltpu.sync_copy(gi_h.at[pl.ds(0, 8), pl.ds(LAST_TILE, 64)], t_gi)

    lane16 = jnp.arange(16, dtype=jnp.int32)
    lane8 = lane16 & 7
    zeros16 = jnp.zeros((16,), jnp.int32)

    def tile_of(u):
        return pl.multiple_of(jnp.minimum(u, LAST_FULL) & -128, 128)

    def fetch(g, carry):
        uvec = idx_u[pl.ds(g * 16, 16)]
        ivec = idx_i[pl.ds(g * 16, 16)]
        for half in range(2):
            us, its, tus, tis = [], [], [], []
            copies = []
            for l in range(8):
                u = uvec[half * 8 + l]
                it = ivec[half * 8 + l]
                tu = tile_of(u)
                ti = tile_of(it)
                us.append(u)
                its.append(it)
                tus.append(tu)
                tis.append(ti)
                copies.append(pltpu.async_copy(
                    mu_h.at[pl.ds(0, 16), pl.ds(tu, 128)], r_mu.at[l], sem))
                copies.append(pltpu.async_copy(
                    mi_h.at[pl.ds(0, 16), pl.ds(ti, 128)], r_mi.at[l], sem))
                copies.append(pltpu.async_copy(
                    gu_h.at[pl.ds(0, 8), pl.ds(tu, 128)], r_gu.at[l], sem))
                copies.append(pltpu.async_copy(
                    gi_h.at[pl.ds(0, 8), pl.ds(ti, 128)], r_gi.at[l], sem))
            for c in copies:
                c.wait()
            for l in range(8):
                u, it, tu, ti = us[l], its[l], tus[l], tis[l]
                e = g * 16 + half * 8 + l
                sl = zeros16 + l
                cu = zeros16 + (u - tu)
                ci = zeros16 + (it - ti)
                cut = zeros16 + jnp.maximum(u - LAST_TILE, 0)
                cit = zeros16 + jnp.maximum(it - LAST_TILE, 0)
                mu_m = plsc.load_gather(r_mu, [sl, lane16, cu])
                mu_t = plsc.load_gather(t_mu, [lane16, cut])
                mu_v[pl.ds(e * 16, 16)] = jnp.where(u >= LAST_TILE, mu_t, mu_m)
                mi_m = plsc.load_gather(r_mi, [sl, lane16, ci])
                mi_t = plsc.load_gather(t_mi, [lane16, cit])
                mi_v[pl.ds(e * 16, 16)] = jnp.where(it >= LAST_TILE, mi_t, mi_m)
                gu_m = plsc.load_gather(r_gu, [sl, lane8, cu])
                gu_t2 = plsc.load_gather(t_gu, [lane8, cut])
                gus = jnp.where(u >= LAST_TILE, gu_t2, gu_m)
                gi_m = plsc.load_gather(r_gi, [sl, lane8, ci])
                gi_t2 = plsc.load_gather(t_gi, [lane8, cit])
                gis = jnp.where(it >= LAST_TILE, gi_t2, gi_m)
                gp_v[pl.ds(e * 16, 16)] = gus * gis
        return carry

    lax.fori_loop(0, BPW // 16, fetch, 0)

    b1v = w_v[pl.ds(48 * 16, 16)]
    b2v = w_v[pl.ds(49 * 16, 16)]
    wlo = w_v[pl.ds(50 * 16, 16)]
    whi = w_v[pl.ds(51 * 16, 16)]

    def elem(b, carry):
        mur = mu_v[pl.ds(b * 16, 16)]
        mir = mi_v[pl.ds(b * 16, 16)]
        acc = b1v
        for i in range(16):
            acc = acc + _bcast(mur, i) * w_v[pl.ds(i * 16, 16)]
        for i in range(16):
            acc = acc + _bcast(mir, i) * w_v[pl.ds((16 + i) * 16, 16)]
        h = jnp.maximum(acc, 0.0)
        acc2 = b2v
        for i in range(16):
            acc2 = acc2 + _bcast(h, i) * w_v[pl.ds((32 + i) * 16, 16)]
        h2 = jnp.maximum(acc2, 0.0)

        flat = b * 16 + lane16
        gpb = gp_v[pl.ds(b * 16, 16)]
        plsc.store_scatter(part_v, [flat], gpb * wlo + h2 * whi)
        return carry

    lax.fori_loop(0, BPW, elem, 0)

    blv = w_v[pl.ds(52 * 16, 16)]
    bl0 = blv[0]
    for g in range(BPW // 16):
        rows = (lane16 + g * 16) * 16
        acc = jnp.zeros((16,), jnp.float32) + bl0
        for c in range(16):
            acc = acc + plsc.load_gather(part_v, [rows + c])
        out_v[pl.ds(g * 16, 16)] = 1.0 / (1.0 + jnp.exp(-acc))

    pltpu.sync_copy(out_v, out_h.at[pl.ds(wid * BPW, BPW)])


@jax.jit
def _fused(user, item, gu_t, gi_t, mu_t, mi_t, params):
    mesh = plsc.VectorSubcoreMesh(core_axis_name="c", subcore_axis_name="s")
    f = functools.partial(
        pl.kernel,
        out_type=jax.ShapeDtypeStruct((BATCH,), jnp.float32),
        mesh=mesh,
        compiler_params=pltpu.CompilerParams(needs_layout_passes=False),
        scratch_types=[
            pltpu.VMEM((BPW,), jnp.int32),
            pltpu.VMEM((BPW,), jnp.int32),
            pltpu.VMEM((BPW * 16,), jnp.float32),
            pltpu.VMEM((BPW * 16,), jnp.float32),
            pltpu.VMEM((BPW * 16,), jnp.float32),
            pltpu.VMEM((53 * 16,), jnp.float32),
            pltpu.VMEM((BPW * 16,), jnp.float32),
            pltpu.VMEM((BPW,), jnp.float32),
            pltpu.VMEM((8, 16, 128), jnp.float32),
            pltpu.VMEM((8, 16, 128), jnp.float32),
            pltpu.VMEM((8, 8, 128), jnp.float32),
            pltpu.VMEM((8, 8, 128), jnp.float32),
            pltpu.VMEM((16, 64), jnp.float32),
            pltpu.VMEM((16, 64), jnp.float32),
            pltpu.VMEM((8, 64), jnp.float32),
            pltpu.VMEM((8, 64), jnp.float32),
            pltpu.SemaphoreType.DMA,
        ],
    )(_body)
    return f(user, item, gu_t, gi_t, mu_t, mi_t, params)


def kernel(user, item, gmf_user_emb, gmf_item_emb, mlp_user_emb, mlp_item_emb,
           W1, b1, W2, b2, Wl, bl):
    gu_t = gmf_user_emb.T
    gi_t = gmf_item_emb.T
    mu_t = mlp_user_emb.T
    mi_t = mlp_item_emb.T
    wl = Wl.reshape(16)
    params = jnp.concatenate([
        W1,
        jnp.pad(W2, ((0, 0), (0, 8))),
        b1.reshape(1, 16),
        jnp.pad(b2, (0, 8)).reshape(1, 16),
        jnp.pad(wl[:8], (0, 8)).reshape(1, 16),
        jnp.pad(wl[8:], (0, 8)).reshape(1, 16),
        jnp.pad(bl, (0, 15)).reshape(1, 16),
    ], axis=0).reshape(-1)
    return _fused(user, item, gu_t, gi_t, mu_t, mi_t, params)


# speedup vs baseline: 8.0165x; 1.4113x over previous
import functools

import jax
import jax.numpy as jnp
from jax import lax
from jax.experimental import pallas as pl
from jax.experimental.pallas import tpu as pltpu
from jax.experimental.pallas import tpu_sc as plsc

BATCH = 16384
NW = 32
BPW = BATCH // NW
NPAIR = BPW // 16
NUSERS = 1000000
LAST_TILE = (NUSERS // 128) * 128
LAST_FULL = LAST_TILE - 128

_BCAST_DNUMS = lax.GatherDimensionNumbers(
    offset_dims=(), collapsed_slice_dims=(0,), start_index_map=(0,))


def _bcast(vec, i):
    idx = jnp.full((16, 1), i, dtype=jnp.int32)
    return lax.gather(vec, idx, _BCAST_DNUMS, (1,),
                      mode=lax.GatherScatterMode.PROMISE_IN_BOUNDS)


def _body(user_h, item_h, gu_h, gi_h, mu_h, mi_h, par_h, out_h,
          idx_u, idx_i, w_v, part_v, out_v,
          r_mu, r_mi, r_gu, r_gi, t_mu, t_mi, t_gu, t_gi, sem0, sem1):
    wid = lax.axis_index("s") * 2 + lax.axis_index("c")

    pltpu.sync_copy(user_h.at[pl.ds(wid * BPW, BPW)], idx_u)
    pltpu.sync_copy(item_h.at[pl.ds(wid * BPW, BPW)], idx_i)
    pltpu.sync_copy(par_h, w_v)
    pltpu.sync_copy(mu_h.at[pl.ds(0, 16), pl.ds(LAST_TILE, 64)], t_mu)
    pltpu.sync_copy(mi_h.at[pl.ds(0, 16), pl.ds(LAST_TILE, 64)], t_mi)
    pltpu.sync_copy(gu_h.at[pl.ds(0, 8), pl.ds(LAST_TILE, 64)], t_gu)
    pltpu.sync_copy(gi_h.at[pl.ds(0, 8), pl.ds(LAST_TILE, 64)], t_gi)

    lane16 = jnp.arange(16, dtype=jnp.int32)
    lane8 = lane16 & 7
    zeros16 = jnp.zeros((16,), jnp.int32)

    b1v = w_v[pl.ds(48 * 16, 16)]
    b2v = w_v[pl.ds(49 * 16, 16)]
    wlo = w_v[pl.ds(50 * 16, 16)]
    whi = w_v[pl.ds(51 * 16, 16)]

    sems = (sem0, sem1)

    def tile_of(u):
        return pl.multiple_of(jnp.minimum(u, LAST_FULL) & -128, 128)

    def issue(uvec, ivec, par, bank):
        sem = sems[bank]
        for l in range(8):
            u = uvec[par * 8 + l]
            it = ivec[par * 8 + l]
            tu = tile_of(u)
            ti = tile_of(it)
            s = bank * 8 + l
            pltpu.async_copy(
                mu_h.at[pl.ds(0, 16), pl.ds(tu, 128)], r_mu.at[s], sem)
            pltpu.async_copy(
                mi_h.at[pl.ds(0, 16), pl.ds(ti, 128)], r_mi.at[s], sem)
            pltpu.async_copy(
                gu_h.at[pl.ds(0, 8), pl.ds(tu, 128)], r_gu.at[s], sem)
            pltpu.async_copy(
                gi_h.at[pl.ds(0, 8), pl.ds(ti, 128)], r_gi.at[s], sem)

    def drain(bank):
        sem = sems[bank]
        for l in range(8):
            s = bank * 8 + l
            pltpu.make_async_copy(
                mu_h.at[pl.ds(0, 16), pl.ds(0, 128)], r_mu.at[s], sem).wait()
            pltpu.make_async_copy(
                mi_h.at[pl.ds(0, 16), pl.ds(0, 128)], r_mi.at[s], sem).wait()
            pltpu.make_async_copy(
                gu_h.at[pl.ds(0, 8), pl.ds(0, 128)], r_gu.at[s], sem).wait()
            pltpu.make_async_copy(
                gi_h.at[pl.ds(0, 8), pl.ds(0, 128)], r_gi.at[s], sem).wait()

    def process(ebase, uvec, ivec, par, bank):
        for l in range(8):
            u = uvec[par * 8 + l]
            it = ivec[par * 8 + l]
            tu = tile_of(u)
            ti = tile_of(it)
            e = ebase + l
            sl = zeros16 + (bank * 8 + l)
            cu = zeros16 + (u - tu)
            ci = zeros16 + (it - ti)
            cut = zeros16 + jnp.maximum(u - LAST_TILE, 0)
            cit = zeros16 + jnp.maximum(it - LAST_TILE, 0)
            mur = jnp.where(u >= LAST_TILE,
                            plsc.load_gather(t_mu, [lane16, cut]),
                            plsc.load_gather(r_mu, [sl, lane16, cu]))
            mir = jnp.where(it >= LAST_TILE,
                            plsc.load_gather(t_mi, [lane16, cit]),
                            plsc.load_gather(r_mi, [sl, lane16, ci]))
            gus = jnp.where(u >= LAST_TILE,
                            plsc.load_gather(t_gu, [lane8, cut]),
                            plsc.load_gather(r_gu, [sl, lane8, cu]))
            gis = jnp.where(it >= LAST_TILE,
                            plsc.load_gather(t_gi, [lane8, cit]),
                            plsc.load_gather(r_gi, [sl, lane8, ci]))
            acc = b1v
            for i in range(16):
                acc = acc + _bcast(mur, i) * w_v[pl.ds(i * 16, 16)]
            for i in range(16):
                acc = acc + _bcast(mir, i) * w_v[pl.ds((16 + i) * 16, 16)]
            h = jnp.maximum(acc, 0.0)
            acc2 = b2v
            for i in range(16):
                acc2 = acc2 + _bcast(h, i) * w_v[pl.ds((32 + i) * 16, 16)]
            h2 = jnp.maximum(acc2, 0.0)
            plsc.store_scatter(part_v, [e * 16 + lane16],
                               gus * gis * wlo + h2 * whi)

    u0 = idx_u[pl.ds(0, 16)]
    i0 = idx_i[pl.ds(0, 16)]
    issue(u0, i0, 0, 0)

    def step(k, carry):
        u_a, i_a = carry
        issue(u_a, i_a, 1, 1)
        drain(0)
        process(k * 16, u_a, i_a, 0, 0)

        def issue_next(_):
            un = idx_u[pl.ds((k + 1) * 16, 16)]
            inn = idx_i[pl.ds((k + 1) * 16, 16)]
            issue(un, inn, 0, 0)
            return un, inn

        u_n, i_n = lax.cond(k < NPAIR - 1, issue_next,
                            lambda _: (u_a, i_a), 0)
        drain(1)
        process(k * 16 + 8, u_a, i_a, 1, 1)
        return u_n, i_n

    lax.fori_loop(0, NPAIR, step, (u0, i0))

    blv = w_v[pl.ds(52 * 16, 16)]
    bl0 = blv[0]
    for g in range(BPW // 16):
        rows = (lane16 + g * 16) * 16
        acc = jnp.zeros((16,), jnp.float32) + bl0
        for c in range(16):
            acc = acc + plsc.load_gather(part_v, [rows + c])
        out_v[pl.ds(g * 16, 16)] = 1.0 / (1.0 + jnp.exp(-acc))

    pltpu.sync_copy(out_v, out_h.at[pl.ds(wid * BPW, BPW)])


@jax.jit
def _fused(user, item, gu_t, gi_t, mu_t, mi_t, params):
    mesh = plsc.VectorSubcoreMesh(core_axis_name="c", subcore_axis_name="s")
    f = functools.partial(
        pl.kernel,
        out_type=jax.ShapeDtypeStruct((BATCH,), jnp.float32),
        mesh=mesh,
        compiler_params=pltpu.CompilerParams(needs_layout_passes=False),
        scratch_types=[
            pltpu.VMEM((BPW,), jnp.int32),
            pltpu.VMEM((BPW,), jnp.int32),
            pltpu.VMEM((53 * 16,), jnp.float32),
            pltpu.VMEM((BPW * 16,), jnp.float32),
            pltpu.VMEM((BPW,), jnp.float32),
            pltpu.VMEM((16, 16, 128), jnp.float32),
            pltpu.VMEM((16, 16, 128), jnp.float32),
            pltpu.VMEM((16, 8, 128), jnp.float32),
            pltpu.VMEM((16, 8, 128), jnp.float32),
            pltpu.VMEM((16, 64), jnp.float32),
            pltpu.VMEM((16, 64), jnp.float32),
            pltpu.VMEM((8, 64), jnp.float32),
            pltpu.VMEM((8, 64), jnp.float32),
            pltpu.SemaphoreType.DMA,
            pltpu.SemaphoreType.DMA,
        ],
    )(_body)
    return f(user, item, gu_t, gi_t, mu_t, mi_t, params)


def kernel(user, item, gmf_user_emb, gmf_item_emb, mlp_user_emb, mlp_item_emb,
           W1, b1, W2, b2, Wl, bl):
    gu_t = gmf_user_emb.T
    gi_t = gmf_item_emb.T
    mu_t = mlp_user_emb.T
    mi_t = mlp_item_emb.T
    wl = Wl.reshape(16)
    params = jnp.concatenate([
        W1,
        jnp.pad(W2, ((0, 0), (0, 8))),
        b1.reshape(1, 16),
        jnp.pad(b2, (0, 8)).reshape(1, 16),
        jnp.pad(wl[:8], (0, 8)).reshape(1, 16),
        jnp.pad(wl[8:], (0, 8)).reshape(1, 16),
        jnp.pad(bl, (0, 15)).reshape(1, 16),
    ], axis=0).reshape(-1)
    return _fused(user, item, gu_t, gi_t, mu_t, mi_t, params)


# speedup vs baseline: 8.0673x; 1.0063x over previous
import functools

import jax
import jax.numpy as jnp
from jax import lax
from jax.experimental import pallas as pl
from jax.experimental.pallas import tpu as pltpu
from jax.experimental.pallas import tpu_sc as plsc

BATCH = 16384
NW = 32
BPW = BATCH // NW
NPAIR = BPW // 16
NUSERS = 1000000
LAST_TILE = (NUSERS // 128) * 128
LAST_FULL = LAST_TILE - 128

_BCAST_DNUMS = lax.GatherDimensionNumbers(
    offset_dims=(), collapsed_slice_dims=(0,), start_index_map=(0,))


def _bcast(vec, i):
    idx = jnp.full((16, 1), i, dtype=jnp.int32)
    return lax.gather(vec, idx, _BCAST_DNUMS, (1,),
                      mode=lax.GatherScatterMode.PROMISE_IN_BOUNDS)


def _body(user_h, item_h, gu_h, gi_h, mu_h, mi_h, par_h, out_h,
          idx_u, idx_i, w_v, part_v, out_v,
          r_mu, r_mi, r_gu, r_gi, t_mu, t_mi, t_gu, t_gi, sem0, sem1):
    wid = lax.axis_index("s") * 2 + lax.axis_index("c")

    pltpu.sync_copy(user_h.at[pl.ds(wid * BPW, BPW)], idx_u)
    pltpu.sync_copy(item_h.at[pl.ds(wid * BPW, BPW)], idx_i)
    pltpu.sync_copy(par_h, w_v)
    pltpu.sync_copy(mu_h.at[pl.ds(0, 16), pl.ds(LAST_TILE, 64)], t_mu)
    pltpu.sync_copy(mi_h.at[pl.ds(0, 16), pl.ds(LAST_TILE, 64)], t_mi)
    pltpu.sync_copy(gu_h.at[pl.ds(0, 8), pl.ds(LAST_TILE, 64)], t_gu)
    pltpu.sync_copy(gi_h.at[pl.ds(0, 8), pl.ds(LAST_TILE, 64)], t_gi)

    lane16 = jnp.arange(16, dtype=jnp.int32)
    lane8 = lane16 & 7
    zeros16 = jnp.zeros((16,), jnp.int32)

    b1v = w_v[pl.ds(48 * 16, 16)]
    b2v = w_v[pl.ds(49 * 16, 16)]
    wlo = w_v[pl.ds(50 * 16, 16)]
    whi = w_v[pl.ds(51 * 16, 16)]

    sems = (sem0, sem1)

    def tile_of(u):
        return pl.multiple_of(jnp.minimum(u, LAST_FULL) & -128, 128)

    def issue(uvec, ivec, par, bank):
        sem = sems[bank]
        for l in range(8):
            u = uvec[par * 8 + l]
            it = ivec[par * 8 + l]
            tu = tile_of(u)
            ti = tile_of(it)
            s = bank * 8 + l
            pltpu.async_copy(
                mu_h.at[pl.ds(0, 16), pl.ds(tu, 128)], r_mu.at[s], sem)
            pltpu.async_copy(
                mi_h.at[pl.ds(0, 16), pl.ds(ti, 128)], r_mi.at[s], sem)
            pltpu.async_copy(
                gu_h.at[pl.ds(0, 8), pl.ds(tu, 128)], r_gu.at[s], sem)
            pltpu.async_copy(
                gi_h.at[pl.ds(0, 8), pl.ds(ti, 128)], r_gi.at[s], sem)

    def drain(bank):
        sem = sems[bank]
        for l in range(8):
            s = bank * 8 + l
            pltpu.make_async_copy(
                mu_h.at[pl.ds(0, 16), pl.ds(0, 128)], r_mu.at[s], sem).wait()
            pltpu.make_async_copy(
                mi_h.at[pl.ds(0, 16), pl.ds(0, 128)], r_mi.at[s], sem).wait()
            pltpu.make_async_copy(
                gu_h.at[pl.ds(0, 8), pl.ds(0, 128)], r_gu.at[s], sem).wait()
            pltpu.make_async_copy(
                gi_h.at[pl.ds(0, 8), pl.ds(0, 128)], r_gi.at[s], sem).wait()

    def process(ebase, uvec, ivec, par, bank):
        for l in range(8):
            u = uvec[par * 8 + l]
            it = ivec[par * 8 + l]
            tu = tile_of(u)
            ti = tile_of(it)
            e = ebase + l
            sl = zeros16 + (bank * 8 + l)
            cu = zeros16 + (u - tu)
            ci = zeros16 + (it - ti)
            cut = zeros16 + jnp.maximum(u - LAST_TILE, 0)
            cit = zeros16 + jnp.maximum(it - LAST_TILE, 0)
            mur = jnp.where(u >= LAST_TILE,
                            plsc.load_gather(t_mu, [lane16, cut]),
                            plsc.load_gather(r_mu, [sl, lane16, cu]))
            mir = jnp.where(it >= LAST_TILE,
                            plsc.load_gather(t_mi, [lane16, cit]),
                            plsc.load_gather(r_mi, [sl, lane16, ci]))
            gus = jnp.where(u >= LAST_TILE,
                            plsc.load_gather(t_gu, [lane8, cut]),
                            plsc.load_gather(r_gu, [sl, lane8, cu]))
            gis = jnp.where(it >= LAST_TILE,
                            plsc.load_gather(t_gi, [lane8, cit]),
                            plsc.load_gather(r_gi, [sl, lane8, ci]))
            acc = b1v
            for i in range(16):
                acc = acc + _bcast(mur, i) * w_v[pl.ds(i * 16, 16)]
            for i in range(16):
                acc = acc + _bcast(mir, i) * w_v[pl.ds((16 + i) * 16, 16)]
            h = jnp.maximum(acc, 0.0)
            acc2 = b2v
            for i in range(16):
                acc2 = acc2 + _bcast(h, i) * w_v[pl.ds((32 + i) * 16, 16)]
            h2 = jnp.maximum(acc2, 0.0)
            plsc.store_scatter(part_v, [e * 16 + lane16],
                               gus * gis * wlo + h2 * whi)

    u0 = idx_u[pl.ds(0, 16)]
    i0 = idx_i[pl.ds(0, 16)]
    issue(u0, i0, 0, 0)

    def step(k, carry):
        u_a, i_a = carry
        issue(u_a, i_a, 1, 1)
        drain(0)
        process(k * 16, u_a, i_a, 0, 0)

        def issue_next(_):
            un = idx_u[pl.ds((k + 1) * 16, 16)]
            inn = idx_i[pl.ds((k + 1) * 16, 16)]
            issue(un, inn, 0, 0)
            return un, inn

        u_n, i_n = lax.cond(k < NPAIR - 1, issue_next,
                            lambda _: (u_a, i_a), 0)
        drain(1)
        process(k * 16 + 8, u_a, i_a, 1, 1)
        return u_n, i_n

    lax.fori_loop(0, NPAIR, step, (u0, i0))

    blv = w_v[pl.ds(52 * 16, 16)]
    bl0 = blv[0]
    for g in range(BPW // 16):
        rows = (lane16 + g * 16) * 16
        acc = jnp.zeros((16,), jnp.float32) + bl0
        for c in range(16):
            acc = acc + plsc.load_gather(part_v, [rows + c])
        out_v[pl.ds(g * 16, 16)] = 1.0 / (1.0 + jnp.exp(-acc))

    pltpu.sync_copy(out_v, out_h.at[pl.ds(wid * BPW, BPW)])


@jax.jit
def _fused(user, item, gu_t, gi_t, mu_t, mi_t, params):
    mesh = plsc.VectorSubcoreMesh(core_axis_name="c", subcore_axis_name="s")
    f = functools.partial(
        pl.kernel,
        out_type=jax.ShapeDtypeStruct((BATCH,), jnp.float32),
        mesh=mesh,
        compiler_params=pltpu.CompilerParams(
            needs_layout_passes=False, skip_device_barrier=True),
        scratch_types=[
            pltpu.VMEM((BPW,), jnp.int32),
            pltpu.VMEM((BPW,), jnp.int32),
            pltpu.VMEM((53 * 16,), jnp.float32),
            pltpu.VMEM((BPW * 16,), jnp.float32),
            pltpu.VMEM((BPW,), jnp.float32),
            pltpu.VMEM((16, 16, 128), jnp.float32),
            pltpu.VMEM((16, 16, 128), jnp.float32),
            pltpu.VMEM((16, 8, 128), jnp.float32),
            pltpu.VMEM((16, 8, 128), jnp.float32),
            pltpu.VMEM((16, 64), jnp.float32),
            pltpu.VMEM((16, 64), jnp.float32),
            pltpu.VMEM((8, 64), jnp.float32),
            pltpu.VMEM((8, 64), jnp.float32),
            pltpu.SemaphoreType.DMA,
            pltpu.SemaphoreType.DMA,
        ],
    )(_body)
    return f(user, item, gu_t, gi_t, mu_t, mi_t, params)


def kernel(user, item, gmf_user_emb, gmf_item_emb, mlp_user_emb, mlp_item_emb,
           W1, b1, W2, b2, Wl, bl):
    gu_t = gmf_user_emb.T
    gi_t = gmf_item_emb.T
    mu_t = mlp_user_emb.T
    mi_t = mlp_item_emb.T
    wl = Wl.reshape(16)
    params = jnp.concatenate([
        W1,
        jnp.pad(W2, ((0, 0), (0, 8))),
        b1.reshape(1, 16),
        jnp.pad(b2, (0, 8)).reshape(1, 16),
        jnp.pad(wl[:8], (0, 8)).reshape(1, 16),
        jnp.pad(wl[8:], (0, 8)).reshape(1, 16),
        jnp.pad(bl, (0, 15)).reshape(1, 16),
    ], axis=0).reshape(-1)
    return _fused(user, item, gu_t, gi_t, mu_t, mi_t, params)
